# baseline (speedup 1.0000x reference)
import functools

import jax
import jax.numpy as jnp
from jax import lax
from jax.experimental import pallas as pl
from jax.experimental.pallas import tpu as pltpu
from jax.experimental.pallas import tpu_sc as plsc

B, C, F = 1024, 1000, 20000

NC, NS, L = 2, 16, 16
NW = NC * NS
CHUNK = 128
PER_W = ((F + NW * CHUNK - 1) // (NW * CHUNK)) * CHUNK
F_PAD = PER_W * NW
N_CHUNKS = PER_W // CHUNK


C2 = 1024


KB = 4
BK = B // KB


def _tc_body(xt_ref, tt_ref, gp_ref, bce_ref, g_acc, s_acc, bce_acc):
    k = pl.program_id(0)
    x = xt_ref[...]
    t = tt_ref[...]
    part = jnp.sum(
        jnp.maximum(x, 0.0) - x * t + jnp.log1p(jnp.exp(-jnp.abs(x)))
    )
    p = jax.nn.sigmoid(x)
    pz = jnp.concatenate([p, jnp.zeros((C2 - C, BK), jnp.float32)], axis=0)
    sp = jnp.sum(pz, axis=1)
    gpart = lax.dot_general(pz, pz, (((1,), (1,)), ((), ())),
                            preferred_element_type=jnp.float32)

    @pl.when(k == 0)
    def _():
        g_acc[...] = gpart
        s_acc[...] = sp
        bce_acc[0] = part

    @pl.when(k > 0)
    def _():
        g_acc[...] += gpart
        s_acc[...] += sp
        bce_acc[0] += part

    @pl.when(k == KB - 1)
    def _():
        bce_ref[0, 0] = bce_acc[0]
        gp_ref[...] = (s_acc[...][:, None] - g_acc[...]).reshape(
            C2 * C2 // 128, 128)


def _tc_stage(x, t):
    return pl.pallas_call(
        _tc_body,
        grid=(KB,),
        in_specs=(
            pl.BlockSpec((C, BK), lambda k: (0, k)),
            pl.BlockSpec((C, BK), lambda k: (0, k)),
        ),
        out_shape=(
            jax.ShapeDtypeStruct((C2 * C2 // 128, 128), jnp.float32),
            jax.ShapeDtypeStruct((1, 1), jnp.float32),
        ),
        out_specs=(
            pl.BlockSpec((C2 * C2 // 128, 128), lambda k: (0, 0)),
            pl.BlockSpec(memory_space=pltpu.SMEM, block_shape=(1, 1),
                         index_map=lambda k: (0, 0)),
        ),
        scratch_shapes=[
            pltpu.VMEM((C2, C2), jnp.float32),
            pltpu.VMEM((C2,), jnp.float32),
            pltpu.SMEM((1,), jnp.float32),
        ],
    )(x, t)


@functools.cache
def _make_sc_stage():
    mesh = plsc.VectorSubcoreMesh(core_axis_name="c", subcore_axis_name="s")

    @functools.partial(
        pl.kernel,
        mesh=mesh,
        out_type=jax.ShapeDtypeStruct((NW, L), jnp.float32),
        scratch_types=[
            pltpu.VMEM((PER_W,), jnp.int32),
            pltpu.VMEM((PER_W,), jnp.int32),
            pltpu.VMEM((N_CHUNKS, CHUNK), jnp.int32),
            pltpu.VMEM((N_CHUNKS, CHUNK), jnp.float32),
            pltpu.VMEM((L,), jnp.float32),
            pltpu.SemaphoreType.DMA,
        ],
    )
    def sc_kernel(fl_hbm, fr_hbm, gp_hbm, out_hbm,
                  fl_v, fr_v, idx_v, g_v, acc_v, sem):
        wid = lax.axis_index("s") * NC + lax.axis_index("c")
        base = wid * PER_W
        rbase = jnp.minimum(base, F - PER_W)
        pltpu.sync_copy(fl_hbm.at[pl.ds(rbase, PER_W)], fl_v)
        pltpu.sync_copy(fr_hbm.at[pl.ds(rbase, PER_W)], fr_v)
        for c in range(N_CHUNKS):
            for i in range(CHUNK // L):
                sl = pl.ds(c * CHUNK + i * L, L)
                idx_v[c, pl.ds(i * L, L)] = fl_v[sl] * C2 + fr_v[sl]
        copies = [
            pltpu.async_copy(gp_hbm.at[idx_v.at[c]], g_v.at[c], sem)
            for c in range(N_CHUNKS)
        ]
        for cp in copies:
            cp.wait()
        acc = jnp.zeros((L,), jnp.float32)
        lane = lax.broadcasted_iota(jnp.int32, (L,), 0)
        for c in range(N_CHUNKS):
            for i in range(CHUNK // L):
                pos = rbase + c * CHUNK + i * L + lane
                v = g_v[c, pl.ds(i * L, L)]
                acc = acc + jnp.where(pos >= base, v, 0.0)
        acc_v[...] = acc
        pltpu.sync_copy(acc_v, out_hbm.at[wid])

    return sc_kernel


def kernel(input, target, filter_l, filter_r):
    gp, bce = _tc_stage(input.T, target.T)
    partials = _make_sc_stage()(filter_l.astype(jnp.int32),
                                filter_r.astype(jnp.int32),
                                gp.reshape(C2 * C2))
    implication = jnp.sum(partials) / B
    return bce[0, 0] / (B * C) + 0.01 * implication


# speedup vs baseline: 1.1236x; 1.1236x over previous
import functools

import jax
import jax.numpy as jnp
from jax import lax
from jax.experimental import pallas as pl
from jax.experimental.pallas import tpu as pltpu
from jax.experimental.pallas import tpu_sc as plsc

B, C, F = 1024, 1000, 20000

NC, NS, L = 2, 16, 16
NW = NC * NS
CHUNK = 128
PER_W = ((F + NW * CHUNK - 1) // (NW * CHUNK)) * CHUNK
N_CHUNKS = PER_W // CHUNK

C2 = 1024


def _tc_body(xt_ref, tt_ref, gp_ref, bce_ref):
    x = xt_ref[...]
    t = tt_ref[...]
    p = jax.nn.sigmoid(x)
    bce_ref[0, 0] = jnp.sum(
        jnp.maximum(x, 0.0) - x * t
        - jnp.log(jnp.where(x >= 0.0, p, 1.0 - p))
    )
    pz = jnp.concatenate([p, jnp.zeros((C2 - C, B), jnp.float32)], axis=0)
    s = jnp.sum(pz, axis=1)
    g = lax.dot_general(pz, pz, (((1,), (1,)), ((), ())),
                        preferred_element_type=jnp.float32)
    gp_ref[...] = (s[:, None] - g).reshape(C2 * C2 // 128, 128)


def _tc_stage(x, t):
    return pl.pallas_call(
        _tc_body,
        out_shape=(
            jax.ShapeDtypeStruct((C2 * C2 // 128, 128), jnp.float32),
            jax.ShapeDtypeStruct((1, 1), jnp.float32),
        ),
        out_specs=(
            pl.BlockSpec(memory_space=pltpu.VMEM),
            pl.BlockSpec(memory_space=pltpu.SMEM),
        ),
    )(x, t)


@functools.cache
def _make_sc_stage():
    mesh = plsc.VectorSubcoreMesh(core_axis_name="c", subcore_axis_name="s")

    @functools.partial(
        pl.kernel,
        mesh=mesh,
        out_type=jax.ShapeDtypeStruct((NW, L), jnp.float32),
        scratch_types=[
            pltpu.VMEM((PER_W,), jnp.int32),
            pltpu.VMEM((PER_W,), jnp.int32),
            pltpu.VMEM((N_CHUNKS, CHUNK), jnp.int32),
            pltpu.VMEM((N_CHUNKS, CHUNK), jnp.float32),
            pltpu.VMEM((L,), jnp.float32),
            pltpu.SemaphoreType.DMA,
        ],
    )
    def sc_kernel(fl_hbm, fr_hbm, gp_hbm, out_hbm,
                  fl_v, fr_v, idx_v, g_v, acc_v, sem):
        wid = lax.axis_index("s") * NC + lax.axis_index("c")
        base = wid * PER_W
        rbase = jnp.minimum(base, F - PER_W)
        pltpu.sync_copy(fl_hbm.at[pl.ds(rbase, PER_W)], fl_v)
        pltpu.sync_copy(fr_hbm.at[pl.ds(rbase, PER_W)], fr_v)
        copies = []
        for c in range(N_CHUNKS):
            for i in range(CHUNK // L):
                sl = pl.ds(c * CHUNK + i * L, L)
                idx_v[c, pl.ds(i * L, L)] = fl_v[sl] * C2 + fr_v[sl]
            copies.append(pltpu.async_copy(gp_hbm.at[idx_v.at[c]],
                                           g_v.at[c], sem))
        acc = jnp.zeros((L,), jnp.float32)
        lane = lax.broadcasted_iota(jnp.int32, (L,), 0)
        for c in range(N_CHUNKS):
            copies[c].wait()
            for i in range(CHUNK // L):
                pos = rbase + c * CHUNK + i * L + lane
                v = g_v[c, pl.ds(i * L, L)]
                acc = acc + jnp.where(pos >= base, v, 0.0)
        acc_v[...] = acc
        pltpu.sync_copy(acc_v, out_hbm.at[wid])

    return sc_kernel


def kernel(input, target, filter_l, filter_r):
    gp, bce = _tc_stage(input.T, target.T)
    partials = _make_sc_stage()(filter_l.astype(jnp.int32),
                                filter_r.astype(jnp.int32),
                                gp.reshape(C2 * C2))
    implication = jnp.sum(partials) / B
    return bce[0, 0] / (B * C) + 0.01 * implication
